# Initial kernel scaffold; baseline (speedup 1.0000x reference)
import jax
import jax.numpy as jnp
from jax.experimental import pallas as pl


def kernel(x):
    raise NotImplementedError("write your pallas kernel here")


# speedup vs baseline: 2.4696x; 2.4696x over previous
import functools

import jax
import jax.numpy as jnp
from jax import lax
from jax.experimental import pallas as pl
from jax.experimental.pallas import tpu as pltpu
from jax.experimental.pallas import tpu_sc as plsc

NUM_BINS = 1024
NC = 2
NS = 16
L = 16
NW = NC * NS

CHUNK = 16384
NBUF = 2


def _hist_body(n_per_tile, x_hbm, part_hbm, buf, hist, *sems):
    wid = lax.axis_index("s") * NC + lax.axis_index("c")
    base = wid * n_per_tile
    n_chunks = n_per_tile // CHUNK

    zeros = jnp.zeros((L,), jnp.int32)
    ones = jnp.ones((L,), jnp.int32)

    @pl.loop(0, NUM_BINS // L)
    def _zero(i):
        hist[pl.ds(i * L, L)] = zeros

    for b in range(NBUF):
        pltpu.async_copy(x_hbm.at[pl.ds(base + b * CHUNK, CHUNK)],
                         buf.at[b], sems[b])

    @pl.loop(0, n_chunks // NBUF)
    def _outer(g):
        c0 = g * NBUF
        for b in range(NBUF):
            c = c0 + b
            pltpu.make_async_copy(x_hbm.at[pl.ds(base + c * CHUNK, CHUNK)],
                                  buf.at[b], sems[b]).wait()

            @pl.loop(0, CHUNK // L)
            def _inner(i):
                idx = buf[b, pl.ds(i * L, L)]
                plsc.addupdate_scatter(hist, [idx], ones)

            nxt = c + NBUF

            @pl.when(nxt < n_chunks)
            def _refill():
                pltpu.async_copy(
                    x_hbm.at[pl.ds(base + nxt * CHUNK, CHUNK)],
                    buf.at[b], sems[b])

    pltpu.sync_copy(hist, part_hbm.at[wid])


@jax.jit
def _sc_hist(x):
    n = x.shape[0]
    n_per_tile = n // NW
    mesh = plsc.VectorSubcoreMesh(core_axis_name="c", subcore_axis_name="s")
    body = functools.partial(_hist_body, n_per_tile)
    f = pl.kernel(
        body,
        out_type=jax.ShapeDtypeStruct((NW, NUM_BINS), jnp.int32),
        mesh=mesh,
        compiler_params=pltpu.CompilerParams(needs_layout_passes=False),
        scratch_types=[
            pltpu.VMEM((NBUF, CHUNK), jnp.int32),
            pltpu.VMEM((NUM_BINS,), jnp.int32),
        ] + [pltpu.SemaphoreType.DMA] * NBUF,
    )
    return f(x)


def _reduce_body(p_ref, o_ref):
    o_ref[...] = jnp.sum(p_ref[...], axis=0, keepdims=True)


@jax.jit
def _reduce(part):
    out = pl.pallas_call(
        _reduce_body,
        out_shape=jax.ShapeDtypeStruct((1, NUM_BINS), jnp.int32),
    )(part)
    return out.reshape(NUM_BINS)


def kernel(x):
    assert x.shape[0] % (NW * CHUNK * NBUF) == 0
    part = _sc_hist(x)
    return _reduce(part)


# speedup vs baseline: 2.4772x; 1.0031x over previous
import functools

import jax
import jax.numpy as jnp
from jax import lax
from jax.experimental import pallas as pl
from jax.experimental.pallas import tpu as pltpu
from jax.experimental.pallas import tpu_sc as plsc

NUM_BINS = 1024
NC = 2
NS = 16
L = 16
NW = NC * NS

CHUNK = 16384
NBUF = 2


def _hist_body(n_per_tile, x_hbm, part_hbm, buf, hist, *sems):
    wid = lax.axis_index("s") * NC + lax.axis_index("c")
    base = wid * n_per_tile
    n_chunks = n_per_tile // CHUNK

    zeros = jnp.zeros((L,), jnp.int32)
    ones = jnp.ones((L,), jnp.int32)

    @pl.loop(0, NUM_BINS // L)
    def _zero(i):
        hist[pl.ds(i * L, L)] = zeros

    for b in range(NBUF):
        pltpu.async_copy(x_hbm.at[pl.ds(base + b * CHUNK, CHUNK)],
                         buf.at[b], sems[b])

    @pl.loop(0, n_chunks // NBUF)
    def _outer(g):
        c0 = g * NBUF
        for b in range(NBUF):
            c = c0 + b
            pltpu.make_async_copy(x_hbm.at[pl.ds(base + c * CHUNK, CHUNK)],
                                  buf.at[b], sems[b]).wait()

            @pl.loop(0, CHUNK // L, unroll=8)
            def _inner(i):
                idx = buf[b, pl.ds(i * L, L)]
                plsc.addupdate_scatter(hist, [idx], ones)

            nxt = c + NBUF

            @pl.when(nxt < n_chunks)
            def _refill():
                pltpu.async_copy(
                    x_hbm.at[pl.ds(base + nxt * CHUNK, CHUNK)],
                    buf.at[b], sems[b])

    pltpu.sync_copy(hist, part_hbm.at[wid])


@jax.jit
def _sc_hist(x):
    n = x.shape[0]
    n_per_tile = n // NW
    mesh = plsc.VectorSubcoreMesh(core_axis_name="c", subcore_axis_name="s")
    body = functools.partial(_hist_body, n_per_tile)
    f = pl.kernel(
        body,
        out_type=jax.ShapeDtypeStruct((NW, NUM_BINS), jnp.int32),
        mesh=mesh,
        compiler_params=pltpu.CompilerParams(needs_layout_passes=False),
        scratch_types=[
            pltpu.VMEM((NBUF, CHUNK), jnp.int32),
            pltpu.VMEM((NUM_BINS,), jnp.int32),
        ] + [pltpu.SemaphoreType.DMA] * NBUF,
    )
    return f(x)


def _reduce_body(p_ref, o_ref):
    o_ref[...] = jnp.sum(p_ref[...], axis=0, keepdims=True)


@jax.jit
def _reduce(part):
    out = pl.pallas_call(
        _reduce_body,
        out_shape=jax.ShapeDtypeStruct((1, NUM_BINS), jnp.int32),
    )(part)
    return out.reshape(NUM_BINS)


def kernel(x):
    assert x.shape[0] % (NW * CHUNK * NBUF) == 0
    part = _sc_hist(x)
    return _reduce(part)
